# baseline (speedup 1.0000x reference)
import jax
import jax.numpy as jnp
from jax.experimental import pallas as pl
from jax.experimental.pallas import tpu as pltpu

_NEI = 1024
_MAXC = 100
_R = 8
_DC = 8
_NCHUNK = 13

import numpy as np

_TINY = np.float32(1.1754943508222875e-38)


def _threefry_bits(x1):
    ks0 = jnp.uint32(0)
    ks1 = jnp.uint32(42)
    ks2 = jnp.uint32(0 ^ 42 ^ 0x1BD11BDA)
    ks = (ks0, ks1, ks2)
    rots = ((13, 15, 26, 6), (17, 29, 16, 24))

    def rotl(v, r):
        return (v << jnp.uint32(r)) | (v >> jnp.uint32(32 - r))

    x1 = x1 + ks1
    x0 = x1
    x1 = x0 ^ rotl(x1, 13)
    for r in (15, 26, 6):
        x0 = x0 + x1
        x1 = x0 ^ rotl(x1, r)
    x0 = x0 + ks[1]
    x1 = x1 + ks[2] + jnp.uint32(1)
    for i in range(1, 5):
        for r in rots[i % 2]:
            x0 = x0 + x1
            x1 = x0 ^ rotl(x1, r)
        x0 = x0 + ks[(i + 1) % 3]
        x1 = x1 + ks[(i + 2) % 3] + jnp.uint32(i + 1)
    return x0 ^ x1


def _body(theta_ref, yt_ref, yt1_ref, lam_ref, z_ref, loss_ref, colsum, acc):
    i = pl.program_id(0)
    nsteps = pl.num_programs(0)

    @pl.when(i == 0)
    def _init():
        colsum[...] = jnp.zeros_like(colsum)
        acc[0] = 0.0
        acc[1] = 0.0

    theta = theta_ref[...]
    recip = 1.0 / theta
    yti = yt_ref[0, 0, :].astype(jnp.int32)

    r_io = jax.lax.broadcasted_iota(jnp.int32, (_R, _DC, _NEI), 0)
    d_io = jax.lax.broadcasted_iota(jnp.int32, (_R, _DC, _NEI), 1)
    j_io = jax.lax.broadcasted_iota(jnp.int32, (_R, _DC, _NEI), 2)
    row0 = i * _R

    def chunk(c, z_acc):
        dglob = d_io + c * _DC
        idx = ((row0 + r_io) * _MAXC + dglob) * _NEI + j_io
        bits = _threefry_bits(idx.astype(jnp.uint32))
        fb = (bits >> jnp.uint32(9)) | jnp.uint32(0x3F800000)
        f = jax.lax.bitcast_convert_type(fb, jnp.float32) - 1.0
        u = jnp.maximum(f + _TINY, _TINY)
        e = -jnp.log(u)
        val = e * recip[:, None, :]
        minv = jnp.min(val, axis=2, keepdims=True)
        hit = (val == minv) & (dglob < yti[:, None, None])
        return z_acc + jnp.sum(hit.astype(jnp.float32), axis=1)

    z = jax.lax.fori_loop(0, _NCHUNK, chunk,
                          jnp.zeros((_R, _NEI), jnp.float32))
    z_ref[...] = z

    theta_log = jnp.maximum(jnp.log(theta), -104.0)
    z_log = jnp.maximum(jnp.log(z), -104.0)
    obj_c = jnp.sum(z * (theta_log + 1.0 - z_log))
    rowsum = jnp.sum(z, axis=1)
    et_c = jnp.sum((yti.astype(jnp.float32) - rowsum) ** 2)
    acc[0] += obj_c
    acc[1] += et_c
    colsum[...] += jnp.sum(z, axis=0, keepdims=True)

    @pl.when(i == nsteps - 1)
    def _fin():
        cs = colsum[...]
        y1 = yt1_ref[...]
        et1 = jnp.sum((y1 - cs) ** 2)
        g = acc[0] - lam_ref[0, 0] * (acc[1] + et1)
        loss_ref[0, 0] = -g


def kernel(theta, yt, yt1, lam):
    n, nei = theta.shape
    nb = n // _R
    yt3 = yt.reshape(nb, 1, _R)
    yt1b = yt1.reshape(1, nei)
    lamb = jnp.asarray(lam, jnp.float32).reshape(1, 1)

    z, loss = pl.pallas_call(
        _body,
        grid=(nb,),
        in_specs=[
            pl.BlockSpec((_R, nei), lambda i: (i, 0)),
            pl.BlockSpec((1, 1, _R), lambda i: (i, 0, 0)),
            pl.BlockSpec((1, nei), lambda i: (0, 0)),
            pl.BlockSpec(memory_space=pltpu.SMEM),
        ],
        out_specs=[
            pl.BlockSpec((_R, nei), lambda i: (i, 0)),
            pl.BlockSpec(memory_space=pltpu.SMEM),
        ],
        out_shape=[
            jax.ShapeDtypeStruct((n, nei), jnp.float32),
            jax.ShapeDtypeStruct((1, 1), jnp.float32),
        ],
        scratch_shapes=[
            pltpu.VMEM((1, nei), jnp.float32),
            pltpu.SMEM((2,), jnp.float32),
        ],
    )(theta, yt3, yt1b, lamb)
    return (loss[0, 0], z)


# speedup vs baseline: 1.3991x; 1.3991x over previous
import jax
import jax.numpy as jnp
import numpy as np
from jax.experimental import pallas as pl
from jax.experimental.pallas import tpu as pltpu

_NEI = 1024
_MAXC = 100
_DS = 32

_TINY = np.float32(1.1754943508222875e-38)


def _threefry_bits(x1):
    ks0 = jnp.uint32(0)
    ks1 = jnp.uint32(42)
    ks2 = jnp.uint32(0 ^ 42 ^ 0x1BD11BDA)
    ks = (ks0, ks1, ks2)
    rots = ((13, 15, 26, 6), (17, 29, 16, 24))

    def rotl(v, r):
        return (v << jnp.uint32(r)) | (v >> jnp.uint32(32 - r))

    x0 = x1
    x1 = x0 ^ rotl(x1, 13)
    for r in (15, 26, 6):
        x0 = x0 + x1
        x1 = x0 ^ rotl(x1, r)
    x0 = x0 + ks[1]
    x1 = x1 + ks[2] + jnp.uint32(1)
    for i in range(1, 5):
        for r in rots[i % 2]:
            x0 = x0 + x1
            x1 = x0 ^ rotl(x1, r)
        x0 = x0 + ks[(i + 1) % 3]
        x1 = x1 + ks[(i + 2) % 3] + jnp.uint32(i + 1)
    return x0 ^ x1


def _body(yt_ref, lam_ref, theta_ref, yt1_ref, z_ref, loss_ref, colsum, acc):
    i = pl.program_id(0)
    nsteps = pl.num_programs(0)

    @pl.when(i == 0)
    def _init():
        colsum[...] = jnp.zeros_like(colsum)
        acc[0] = 0.0
        acc[1] = 0.0

    cnt = yt_ref[i]
    theta = theta_ref[0]
    recip = jnp.broadcast_to(1.0 / theta, (_DS, _NEI))

    s_io = jax.lax.broadcasted_iota(jnp.int32, (_DS, _NEI), 0)
    j_io = jax.lax.broadcasted_iota(jnp.int32, (_DS, _NEI), 1)
    idx_base = i * (_MAXC * _NEI) + s_io * _NEI + j_io + 42
    s1_io = jax.lax.broadcasted_iota(jnp.int32, (_DS, 1), 0)

    def it_body(it, z8):
        x1 = (idx_base + it * (_DS * _NEI)).astype(jnp.uint32)
        bits = _threefry_bits(x1)
        fb = (bits >> jnp.uint32(9)) | jnp.uint32(0x3F800000)
        f = jax.lax.bitcast_convert_type(fb, jnp.float32) - 1.0
        u = f + _TINY
        val = jnp.log(u) * recip
        maxv = jnp.max(val, axis=1, keepdims=True)
        msel = jnp.where(s1_io < cnt - it * _DS, maxv, 1.0)
        return z8 + jnp.where(val == msel, 1.0, 0.0)

    niter = (cnt + (_DS - 1)) // _DS
    z8 = jax.lax.fori_loop(0, niter, it_body,
                           jnp.zeros((_DS, _NEI), jnp.float32))
    z = jnp.sum(z8, axis=0, keepdims=True)
    z_ref[...] = z.reshape(1, 1, _NEI)

    theta_log = jnp.maximum(jnp.log(theta), -104.0)
    z_log = jnp.maximum(jnp.log(z), -104.0)
    obj_c = jnp.sum(z * (theta_log + 1.0 - z_log))
    et_c = (cnt.astype(jnp.float32) - jnp.sum(z)) ** 2
    acc[0] += obj_c
    acc[1] += et_c
    colsum[...] += z

    @pl.when(i == nsteps - 1)
    def _fin():
        et1 = jnp.sum((yt1_ref[...] - colsum[...]) ** 2)
        g = acc[0] - lam_ref[0, 0] * (acc[1] + et1)
        loss_ref[0, 0] = -g


def kernel(theta, yt, yt1, lam):
    n, nei = theta.shape
    theta3 = theta.reshape(n, 1, nei)
    yti = yt.astype(jnp.int32)
    yt1b = yt1.reshape(1, nei)
    lamb = jnp.asarray(lam, jnp.float32).reshape(1, 1)

    z, loss = pl.pallas_call(
        _body,
        grid=(n,),
        in_specs=[
            pl.BlockSpec(memory_space=pltpu.SMEM),
            pl.BlockSpec(memory_space=pltpu.SMEM),
            pl.BlockSpec((1, 1, nei), lambda i: (i, 0, 0)),
            pl.BlockSpec((1, nei), lambda i: (0, 0)),
        ],
        out_specs=[
            pl.BlockSpec((1, 1, nei), lambda i: (i, 0, 0)),
            pl.BlockSpec(memory_space=pltpu.SMEM),
        ],
        out_shape=[
            jax.ShapeDtypeStruct((n, 1, nei), jnp.float32),
            jax.ShapeDtypeStruct((1, 1), jnp.float32),
        ],
        scratch_shapes=[
            pltpu.VMEM((1, nei), jnp.float32),
            pltpu.SMEM((2,), jnp.float32),
        ],
    )(yti, lamb, theta3, yt1b)
    return (loss[0, 0], z.reshape(n, nei))
